# Initial kernel scaffold; baseline (speedup 1.0000x reference)
import jax
import jax.numpy as jnp
from jax.experimental import pallas as pl


def kernel(x, edge_index, W1l, W1r, b1, W2l, W2r, b2, M1, bm1, M2, bm2):
    raise NotImplementedError("write your pallas kernel here")


# speedup vs baseline: 4.7792x; 4.7792x over previous
import functools

import jax
import jax.numpy as jnp
from jax import lax
from jax.experimental import pallas as pl
from jax.experimental.pallas import tpu as pltpu
from jax.experimental.pallas import tpu_sc as plsc

N = 10000
E = 320000
D = 128
H = 128
C = 64

NC, NS, L = 2, 16, 16
CH = 80
NCHUNK = E // CH
CH_PER_CORE = NCHUNK // NC
TILE_CHUNKS = CH_PER_CORE // NS
NPAD = 10240
RPT = NPAD // NS
DEGW = 16
DROWS = NPAD // D


def _make_seg_sum(with_deg: bool):
    out_type = [jax.ShapeDtypeStruct((NPAD, D), jnp.float32)] * NC
    scratch = [
        pltpu.VMEM((1, CH), jnp.int32),
        pltpu.VMEM((1, CH), jnp.int32),
        pltpu.VMEM((CH, D), jnp.float32),
        pltpu.VMEM_SHARED((NPAD, D), jnp.float32),
        pltpu.SemaphoreType.DMA,
    ]
    if with_deg:
        out_type += [jax.ShapeDtypeStruct((NPAD, D), jnp.float32)] * NC
        scratch += [
            pltpu.VMEM((DROWS, D), jnp.float32),
            pltpu.VMEM((1, DROWS), jnp.int32),
            pltpu.VMEM((DROWS // NS, D), jnp.float32),
            pltpu.VMEM_SHARED((DROWS, D), jnp.float32),
        ]

    mesh = plsc.VectorSubcoreMesh(core_axis_name="c", subcore_axis_name="s")

    @functools.partial(
        pl.kernel, out_type=tuple(out_type), mesh=mesh,
        scratch_types=tuple(scratch),
        compiler_params=pltpu.CompilerParams(needs_layout_passes=False),
    )
    def seg_sum(*refs):
        if with_deg:
            (tbl, src_hbm, dst_hbm, out0, out1, deg0, deg1,
             src_v, dst_v, rows_v, acc_sh, sem,
             deg_v, didx_v, degb_v, deg_sh) = refs
        else:
            (tbl, src_hbm, dst_hbm, out0, out1,
             src_v, dst_v, rows_v, acc_sh, sem) = refs
        c = lax.axis_index("c")
        s = lax.axis_index("s")
        zvec = jnp.zeros((L,), jnp.float32)

        def zrow(i, _):
            for j in range(D // L):
                rows_v[i, pl.ds(j * L, L)] = zvec
            return 0
        lax.fori_loop(0, CH, zrow, 0)
        r0 = s * RPT
        for k in range(RPT // CH):
            pltpu.sync_copy(rows_v, acc_sh.at[pl.ds(r0 + k * CH, CH)])
        if with_deg:
            def dz(i, _):
                for j in range(D // L):
                    deg_v[i, pl.ds(j * L, L)] = zvec
                return 0
            lax.fori_loop(0, DROWS, dz, 0)
            iota = lax.iota(jnp.int32, L)
            for g in range(DROWS // L):
                didx_v[0, pl.ds(g * L, L)] = iota + g * L
            @pl.when(s == 0)
            def _():
                pltpu.sync_copy(rows_v, deg_sh.at[pl.ds(0, CH)])

        plsc.subcore_barrier()

        base = c * CH_PER_CORE + s * TILE_CHUNKS

        ones16 = jnp.ones((L,), jnp.float32)

        def chunk_body(k, _):
            pltpu.sync_copy(src_hbm.at[base + k], src_v)
            pltpu.sync_copy(dst_hbm.at[base + k], dst_v)
            pltpu.async_copy(tbl.at[src_v.at[0]], rows_v, sem).wait()
            pltpu.sync_copy(rows_v, acc_sh.at[dst_v.at[0]], add=True)
            if with_deg:
                for g in range(CH // L):
                    dvec = dst_v[0, pl.ds(g * L, L)]
                    row = lax.shift_right_logical(dvec, 7)
                    col = lax.bitwise_and(dvec, 127)
                    plsc.addupdate_scatter(deg_v, [row, col], ones16)
            return 0
        lax.fori_loop(0, TILE_CHUNKS, chunk_body, 0)
        if with_deg:
            pltpu.sync_copy(deg_v, deg_sh.at[didx_v.at[0]], add=True)
        plsc.subcore_barrier()

        def writeback(o_ref, d_ref):
            for k in range(RPT // CH):
                rr = r0 + k * CH
                pltpu.sync_copy(acc_sh.at[pl.ds(rr, CH)], rows_v)
                pltpu.sync_copy(rows_v, o_ref.at[pl.ds(rr, CH)])
            if with_deg:
                pltpu.sync_copy(deg_sh.at[pl.ds(s * (DROWS // NS),
                                                DROWS // NS)], degb_v)
                for k in range(RPT // CH):
                    def egrp(m, _):
                        j0 = k * CH + m * L
                        row = lax.shift_right_logical(j0, 7)
                        col = lax.bitwise_and(j0, 127)
                        dvec = degb_v[row, pl.ds(col, L)]
                        for l in range(L):
                            vv = jnp.full((L,), dvec[l], jnp.float32)
                            for g in range(D // L):
                                rows_v[m * L + l, pl.ds(g * L, L)] = vv
                        return 0
                    lax.fori_loop(0, CH // L, egrp, 0)
                    pltpu.sync_copy(
                        rows_v, d_ref.at[pl.ds(r0 + k * CH, CH)])

        @pl.when(c == 0)
        def _():
            writeback(out0, deg0 if with_deg else None)

        @pl.when(c == 1)
        def _():
            writeback(out1, deg1 if with_deg else None)

    return seg_sum


_seg_sum_deg = _make_seg_sum(True)
_seg_sum = _make_seg_sum(False)

BLK = 512


def _dotT(a, w):
    return lax.dot_general(a, w, (((1,), (1,)), ((), ())),
                           preferred_element_type=jnp.float32)


def _mean_block(p0_ref, p1_ref, d0_ref, d1_ref):
    psum = p0_ref[...] + p1_ref[...]
    deg = d0_ref[...] + d1_ref[...]
    return psum / jnp.maximum(deg, 1.0)


def _tc1_body(p0_ref, p1_ref, d0_ref, d1_ref, x_ref, wl_ref, wr_ref, b_ref,
              o_ref):
    mean = _mean_block(p0_ref, p1_ref, d0_ref, d1_ref)
    h = _dotT(mean, wl_ref[...]) + _dotT(x_ref[...], wr_ref[...]) + b_ref[...]
    o_ref[...] = jnp.maximum(h, 0.0)


def _tc2_body(p0_ref, p1_ref, d0_ref, d1_ref, h_ref, wl_ref, wr_ref, b_ref,
              m1_ref, bm1_ref, m2_ref, bm2_ref, o_ref):
    mean = _mean_block(p0_ref, p1_ref, d0_ref, d1_ref)
    h = _dotT(mean, wl_ref[...]) + _dotT(h_ref[...], wr_ref[...]) + b_ref[...]
    h = jnp.maximum(h, 0.0)
    h = jnp.maximum(_dotT(h, m1_ref[...]) + bm1_ref[...], 0.0)
    z = _dotT(h, m2_ref[...]) + bm2_ref[...]
    o_ref[...] = 1.0 / (1.0 + jnp.exp(-z))


def _rows(width):
    return pl.BlockSpec((BLK, width), lambda i: (i, 0))


def _full(shape):
    return pl.BlockSpec(shape, lambda i: tuple(0 for _ in shape))


def _tc1(p0, p1, d0, d1, x, Wl, Wr, b):
    return pl.pallas_call(
        _tc1_body,
        grid=(NPAD // BLK,),
        in_specs=[
            _rows(D), _rows(D), _rows(D), _rows(D), _rows(D),
            _full((H, D)), _full((H, D)), _full((1, H)),
        ],
        out_specs=_rows(H),
        out_shape=jax.ShapeDtypeStruct((NPAD, H), jnp.float32),
    )(p0, p1, d0, d1, x, Wl, Wr, b.reshape(1, H))


def _tc2(p0, p1, d0, d1, h1, Wl, Wr, b, M1, bm1, M2, bm2):
    return pl.pallas_call(
        _tc2_body,
        grid=(NPAD // BLK,),
        in_specs=[
            _rows(H), _rows(H), _rows(D), _rows(D), _rows(H),
            _full((H, H)), _full((H, H)), _full((1, H)),
            _full((H, H)), _full((1, H)), _full((C, H)), _full((1, C)),
        ],
        out_specs=_rows(C),
        out_shape=jax.ShapeDtypeStruct((NPAD, C), jnp.float32),
    )(p0, p1, d0, d1, h1, Wl, Wr, b.reshape(1, H), M1, bm1.reshape(1, H),
      M2, bm2.reshape(1, C))


def kernel(x, edge_index, W1l, W1r, b1, W2l, W2r, b2, M1, bm1, M2, bm2):
    src = edge_index[0].reshape(NCHUNK, 1, CH)
    dst = edge_index[1].reshape(NCHUNK, 1, CH)
    xp = jnp.concatenate([x, jnp.zeros((NPAD - N, D), jnp.float32)], axis=0)
    p0, p1, d0, d1 = _seg_sum_deg(xp, src, dst)
    h1 = _tc1(p0, p1, d0, d1, xp, W1l, W1r, b1)
    q0, q1 = _seg_sum(h1, src, dst)
    out = _tc2(q0, q1, d0, d1, h1, W2l, W2r, b2, M1, bm1, M2, bm2)
    return out[:N]


# speedup vs baseline: 6.7253x; 1.4072x over previous
import functools

import jax
import jax.numpy as jnp
from jax import lax
from jax.experimental import pallas as pl
from jax.experimental.pallas import tpu as pltpu
from jax.experimental.pallas import tpu_sc as plsc

N = 10000
E = 320000
D = 128
H = 128
C = 64

NC, NS, L = 2, 16, 16
CH = 80
NCHUNK = E // CH
CH_PER_CORE = NCHUNK // NC
TILE_CHUNKS = CH_PER_CORE // NS
NPAD = 10240
RPT = NPAD // NS
DEGW = 16
DROWS = NPAD // D
SLAB = 25


def _make_seg_sum(with_deg: bool):
    out_type = [jax.ShapeDtypeStruct((NPAD, D), jnp.float32)] * NC
    scratch = [
        pltpu.VMEM((SLAB, 1, CH), jnp.int32),
        pltpu.VMEM((SLAB, 1, CH), jnp.int32),
        pltpu.VMEM((CH, D), jnp.float32),
        pltpu.VMEM_SHARED((NPAD, D), jnp.float32),
        pltpu.SemaphoreType.DMA,
    ]
    if with_deg:
        out_type += [jax.ShapeDtypeStruct((NPAD, D), jnp.float32)] * NC
        scratch += [
            pltpu.VMEM((DROWS, D), jnp.float32),
            pltpu.VMEM((1, DROWS), jnp.int32),
            pltpu.VMEM((DROWS // NS, D), jnp.float32),
            pltpu.VMEM_SHARED((DROWS, D), jnp.float32),
        ]

    mesh = plsc.VectorSubcoreMesh(core_axis_name="c", subcore_axis_name="s")

    @functools.partial(
        pl.kernel, out_type=tuple(out_type), mesh=mesh,
        scratch_types=tuple(scratch),
        compiler_params=pltpu.CompilerParams(needs_layout_passes=False),
    )
    def seg_sum(*refs):
        if with_deg:
            (tbl, src_hbm, dst_hbm, out0, out1, deg0, deg1,
             src_v, dst_v, rows_v, acc_sh, sem,
             deg_v, didx_v, degb_v, deg_sh) = refs
        else:
            (tbl, src_hbm, dst_hbm, out0, out1,
             src_v, dst_v, rows_v, acc_sh, sem) = refs
        c = lax.axis_index("c")
        s = lax.axis_index("s")
        zvec = jnp.zeros((L,), jnp.float32)

        def zrow(i, _):
            for j in range(D // L):
                rows_v[i, pl.ds(j * L, L)] = zvec
            return 0
        lax.fori_loop(0, CH, zrow, 0)
        r0 = s * RPT
        for k in range(RPT // CH):
            pltpu.sync_copy(rows_v, acc_sh.at[pl.ds(r0 + k * CH, CH)])
        if with_deg:
            def dz(i, _):
                for j in range(D // L):
                    deg_v[i, pl.ds(j * L, L)] = zvec
                return 0
            lax.fori_loop(0, DROWS, dz, 0)
            iota = lax.iota(jnp.int32, L)
            for g in range(DROWS // L):
                didx_v[0, pl.ds(g * L, L)] = iota + g * L
            @pl.when(s == 0)
            def _():
                pltpu.sync_copy(rows_v, deg_sh.at[pl.ds(0, CH)])

        plsc.subcore_barrier()

        base = c * CH_PER_CORE + s * TILE_CHUNKS

        ones16 = jnp.ones((L,), jnp.float32)

        for sl in range(TILE_CHUNKS // SLAB):
            pltpu.sync_copy(src_hbm.at[pl.ds(base + sl * SLAB, SLAB)], src_v)
            pltpu.sync_copy(dst_hbm.at[pl.ds(base + sl * SLAB, SLAB)], dst_v)

            def chunk_body(j, _):
                pltpu.async_copy(tbl.at[src_v.at[j, 0]], rows_v, sem).wait()
                pltpu.sync_copy(rows_v, acc_sh.at[dst_v.at[j, 0]], add=True)
                if with_deg:
                    for g in range(CH // L):
                        dvec = dst_v[j, 0, pl.ds(g * L, L)]
                        row = lax.shift_right_logical(dvec, 7)
                        col = lax.bitwise_and(dvec, 127)
                        plsc.addupdate_scatter(deg_v, [row, col], ones16)
                return 0
            lax.fori_loop(0, SLAB, chunk_body, 0)
        if with_deg:
            pltpu.sync_copy(deg_v, deg_sh.at[didx_v.at[0]], add=True)
        plsc.subcore_barrier()

        def writeback(o_ref, d_ref):
            for k in range(RPT // CH):
                rr = r0 + k * CH
                pltpu.sync_copy(acc_sh.at[pl.ds(rr, CH)], rows_v)
                pltpu.sync_copy(rows_v, o_ref.at[pl.ds(rr, CH)])
            if with_deg:
                pltpu.sync_copy(deg_sh.at[pl.ds(s * (DROWS // NS),
                                                DROWS // NS)], degb_v)
                for k in range(RPT // CH):
                    def egrp(m, _):
                        j0 = k * CH + m * L
                        row = lax.shift_right_logical(j0, 7)
                        col = lax.bitwise_and(j0, 127)
                        dvec = degb_v[row, pl.ds(col, L)]
                        for l in range(L):
                            vv = jnp.full((L,), dvec[l], jnp.float32)
                            for g in range(D // L):
                                rows_v[m * L + l, pl.ds(g * L, L)] = vv
                        return 0
                    lax.fori_loop(0, CH // L, egrp, 0)
                    pltpu.sync_copy(
                        rows_v, d_ref.at[pl.ds(r0 + k * CH, CH)])

        @pl.when(c == 0)
        def _():
            writeback(out0, deg0 if with_deg else None)

        @pl.when(c == 1)
        def _():
            writeback(out1, deg1 if with_deg else None)

    return seg_sum


_seg_sum_deg = _make_seg_sum(True)
_seg_sum = _make_seg_sum(False)

BLK = 512


def _dotT(a, w):
    return lax.dot_general(a, w, (((1,), (1,)), ((), ())),
                           preferred_element_type=jnp.float32)


def _mean_block(p0_ref, p1_ref, d0_ref, d1_ref):
    psum = p0_ref[...] + p1_ref[...]
    deg = d0_ref[...] + d1_ref[...]
    return psum / jnp.maximum(deg, 1.0)


def _tc1_body(p0_ref, p1_ref, d0_ref, d1_ref, x_ref, wl_ref, wr_ref, b_ref,
              o_ref):
    mean = _mean_block(p0_ref, p1_ref, d0_ref, d1_ref)
    h = _dotT(mean, wl_ref[...]) + _dotT(x_ref[...], wr_ref[...]) + b_ref[...]
    o_ref[...] = jnp.maximum(h, 0.0)


def _tc2_body(p0_ref, p1_ref, d0_ref, d1_ref, h_ref, wl_ref, wr_ref, b_ref,
              m1_ref, bm1_ref, m2_ref, bm2_ref, o_ref):
    mean = _mean_block(p0_ref, p1_ref, d0_ref, d1_ref)
    h = _dotT(mean, wl_ref[...]) + _dotT(h_ref[...], wr_ref[...]) + b_ref[...]
    h = jnp.maximum(h, 0.0)
    h = jnp.maximum(_dotT(h, m1_ref[...]) + bm1_ref[...], 0.0)
    z = _dotT(h, m2_ref[...]) + bm2_ref[...]
    o_ref[...] = 1.0 / (1.0 + jnp.exp(-z))


def _rows(width):
    return pl.BlockSpec((BLK, width), lambda i: (i, 0))


def _full(shape):
    return pl.BlockSpec(shape, lambda i: tuple(0 for _ in shape))


def _tc1(p0, p1, d0, d1, x, Wl, Wr, b):
    return pl.pallas_call(
        _tc1_body,
        grid=(NPAD // BLK,),
        in_specs=[
            _rows(D), _rows(D), _rows(D), _rows(D), _rows(D),
            _full((H, D)), _full((H, D)), _full((1, H)),
        ],
        out_specs=_rows(H),
        out_shape=jax.ShapeDtypeStruct((NPAD, H), jnp.float32),
    )(p0, p1, d0, d1, x, Wl, Wr, b.reshape(1, H))


def _tc2(p0, p1, d0, d1, h1, Wl, Wr, b, M1, bm1, M2, bm2):
    return pl.pallas_call(
        _tc2_body,
        grid=(NPAD // BLK,),
        in_specs=[
            _rows(H), _rows(H), _rows(D), _rows(D), _rows(H),
            _full((H, H)), _full((H, H)), _full((1, H)),
            _full((H, H)), _full((1, H)), _full((C, H)), _full((1, C)),
        ],
        out_specs=_rows(C),
        out_shape=jax.ShapeDtypeStruct((NPAD, C), jnp.float32),
    )(p0, p1, d0, d1, h1, Wl, Wr, b.reshape(1, H), M1, bm1.reshape(1, H),
      M2, bm2.reshape(1, C))


def kernel(x, edge_index, W1l, W1r, b1, W2l, W2r, b2, M1, bm1, M2, bm2):
    src = edge_index[0].reshape(NCHUNK, 1, CH)
    dst = edge_index[1].reshape(NCHUNK, 1, CH)
    xp = jnp.concatenate([x, jnp.zeros((NPAD - N, D), jnp.float32)], axis=0)
    p0, p1, d0, d1 = _seg_sum_deg(xp, src, dst)
    h1 = _tc1(p0, p1, d0, d1, xp, W1l, W1r, b1)
    q0, q1 = _seg_sum(h1, src, dst)
    out = _tc2(q0, q1, d0, d1, h1, W2l, W2r, b2, M1, bm1, M2, bm2)
    return out[:N]


# speedup vs baseline: 8.3496x; 1.2415x over previous
import functools

import jax
import jax.numpy as jnp
from jax import lax
from jax.experimental import pallas as pl
from jax.experimental.pallas import tpu as pltpu
from jax.experimental.pallas import tpu_sc as plsc

N = 10000
E = 320000
D = 128
H = 128
C = 64

NC, NS, L = 2, 16, 16
CH = 80
NCHUNK = E // CH
CH_PER_CORE = NCHUNK // NC
TILE_CHUNKS = CH_PER_CORE // NS
NPAD = 10240
RPT = NPAD // NS
DEGW = 16
DROWS = NPAD // D
SLAB = 25


def _make_seg_sum(with_deg: bool):
    out_type = [jax.ShapeDtypeStruct((NPAD, D), jnp.float32)] * NC
    scratch = [
        pltpu.VMEM((SLAB, 1, CH), jnp.int32),
        pltpu.VMEM((SLAB, 1, CH), jnp.int32),
        pltpu.VMEM((CH, D), jnp.float32),
        pltpu.VMEM((CH, D), jnp.float32),
        pltpu.VMEM_SHARED((NPAD, D), jnp.float32),
        pltpu.SemaphoreType.DMA,
        pltpu.SemaphoreType.DMA,
        pltpu.SemaphoreType.DMA,
        pltpu.SemaphoreType.DMA,
    ]
    if with_deg:
        out_type += [jax.ShapeDtypeStruct((NPAD, D), jnp.float32)] * NC
        scratch += [
            pltpu.VMEM((DROWS, D), jnp.float32),
            pltpu.VMEM((1, DROWS), jnp.int32),
            pltpu.VMEM((DROWS // NS, D), jnp.float32),
            pltpu.VMEM_SHARED((DROWS, D), jnp.float32),
        ]

    mesh = plsc.VectorSubcoreMesh(core_axis_name="c", subcore_axis_name="s")

    @functools.partial(
        pl.kernel, out_type=tuple(out_type), mesh=mesh,
        scratch_types=tuple(scratch),
        compiler_params=pltpu.CompilerParams(needs_layout_passes=False),
    )
    def seg_sum(*refs):
        if with_deg:
            (tbl, src_hbm, dst_hbm, out0, out1, deg0, deg1,
             src_v, dst_v, rows_v, rows_b, acc_sh, sga, sgb, ssa, ssb,
             deg_v, didx_v, degb_v, deg_sh) = refs
        else:
            (tbl, src_hbm, dst_hbm, out0, out1,
             src_v, dst_v, rows_v, rows_b, acc_sh, sga, sgb, ssa, ssb) = refs
        c = lax.axis_index("c")
        s = lax.axis_index("s")
        zvec = jnp.zeros((L,), jnp.float32)

        def zrow(i, _):
            for j in range(D // L):
                rows_v[i, pl.ds(j * L, L)] = zvec
            return 0
        lax.fori_loop(0, CH, zrow, 0)
        r0 = s * RPT
        for k in range(RPT // CH):
            pltpu.sync_copy(rows_v, acc_sh.at[pl.ds(r0 + k * CH, CH)])
        if with_deg:
            def dz(i, _):
                for j in range(D // L):
                    deg_v[i, pl.ds(j * L, L)] = zvec
                return 0
            lax.fori_loop(0, DROWS, dz, 0)
            iota = lax.iota(jnp.int32, L)
            for g in range(DROWS // L):
                didx_v[0, pl.ds(g * L, L)] = iota + g * L
            @pl.when(s == 0)
            def _():
                pltpu.sync_copy(rows_v, deg_sh.at[pl.ds(0, CH)])

        plsc.subcore_barrier()

        base = c * CH_PER_CORE + s * TILE_CHUNKS

        ones16 = jnp.ones((L,), jnp.float32)

        def hist(j):
            if with_deg:
                for g in range(CH // L):
                    dvec = dst_v[j, 0, pl.ds(g * L, L)]
                    row = lax.shift_right_logical(dvec, 7)
                    col = lax.bitwise_and(dvec, 127)
                    plsc.addupdate_scatter(deg_v, [row, col], ones16)

        def start_g(j, buf, gs):
            pltpu.async_copy(tbl.at[src_v.at[j, 0]], buf, gs)

        def wait_g(j, buf, gs):
            pltpu.make_async_copy(tbl.at[src_v.at[j, 0]], buf, gs).wait()

        def start_s(j, buf, ss):
            pltpu.async_copy(buf, acc_sh.at[dst_v.at[j, 0]], ss, add=True)

        def wait_s(j, buf, ss):
            pltpu.make_async_copy(buf, acc_sh.at[dst_v.at[j, 0]], ss).wait()

        for sl in range(TILE_CHUNKS // SLAB):
            pltpu.sync_copy(src_hbm.at[pl.ds(base + sl * SLAB, SLAB)], src_v)
            pltpu.sync_copy(dst_hbm.at[pl.ds(base + sl * SLAB, SLAB)], dst_v)
            start_g(0, rows_v, sga)

            def pair_body(m, _):
                k0 = 2 * m
                wait_g(k0, rows_v, sga)

                @pl.when(k0 + 1 < SLAB)
                def _():
                    start_g(k0 + 1, rows_b, sgb)
                start_s(k0, rows_v, ssa)
                hist(k0)
                wait_s(k0, rows_v, ssa)

                @pl.when(k0 + 1 < SLAB)
                def _():
                    k1 = k0 + 1
                    wait_g(k1, rows_b, sgb)

                    @pl.when(k1 + 1 < SLAB)
                    def _():
                        start_g(k1 + 1, rows_v, sga)
                    start_s(k1, rows_b, ssb)
                    hist(k1)
                    wait_s(k1, rows_b, ssb)
                return 0
            lax.fori_loop(0, (SLAB + 1) // 2, pair_body, 0)
        if with_deg:
            pltpu.sync_copy(deg_v, deg_sh.at[didx_v.at[0]], add=True)
        plsc.subcore_barrier()

        def writeback(o_ref, d_ref):
            for k in range(RPT // CH):
                rr = r0 + k * CH
                pltpu.sync_copy(acc_sh.at[pl.ds(rr, CH)], rows_v)
                pltpu.sync_copy(rows_v, o_ref.at[pl.ds(rr, CH)])
            if with_deg:
                pltpu.sync_copy(deg_sh.at[pl.ds(s * (DROWS // NS),
                                                DROWS // NS)], degb_v)
                for k in range(RPT // CH):
                    def egrp(m, _):
                        j0 = k * CH + m * L
                        row = lax.shift_right_logical(j0, 7)
                        col = lax.bitwise_and(j0, 127)
                        dvec = degb_v[row, pl.ds(col, L)]
                        for l in range(L):
                            vv = jnp.full((L,), dvec[l], jnp.float32)
                            for g in range(D // L):
                                rows_v[m * L + l, pl.ds(g * L, L)] = vv
                        return 0
                    lax.fori_loop(0, CH // L, egrp, 0)
                    pltpu.sync_copy(
                        rows_v, d_ref.at[pl.ds(r0 + k * CH, CH)])

        @pl.when(c == 0)
        def _():
            writeback(out0, deg0 if with_deg else None)

        @pl.when(c == 1)
        def _():
            writeback(out1, deg1 if with_deg else None)

    return seg_sum


_seg_sum_deg = _make_seg_sum(True)
_seg_sum = _make_seg_sum(False)

BLK = 512


def _dotT(a, w):
    return lax.dot_general(a, w, (((1,), (1,)), ((), ())),
                           preferred_element_type=jnp.float32)


def _mean_block(p0_ref, p1_ref, d0_ref, d1_ref):
    psum = p0_ref[...] + p1_ref[...]
    deg = d0_ref[...] + d1_ref[...]
    return psum / jnp.maximum(deg, 1.0)


def _tc1_body(p0_ref, p1_ref, d0_ref, d1_ref, x_ref, wl_ref, wr_ref, b_ref,
              o_ref):
    mean = _mean_block(p0_ref, p1_ref, d0_ref, d1_ref)
    h = _dotT(mean, wl_ref[...]) + _dotT(x_ref[...], wr_ref[...]) + b_ref[...]
    o_ref[...] = jnp.maximum(h, 0.0)


def _tc2_body(p0_ref, p1_ref, d0_ref, d1_ref, h_ref, wl_ref, wr_ref, b_ref,
              m1_ref, bm1_ref, m2_ref, bm2_ref, o_ref):
    mean = _mean_block(p0_ref, p1_ref, d0_ref, d1_ref)
    h = _dotT(mean, wl_ref[...]) + _dotT(h_ref[...], wr_ref[...]) + b_ref[...]
    h = jnp.maximum(h, 0.0)
    h = jnp.maximum(_dotT(h, m1_ref[...]) + bm1_ref[...], 0.0)
    z = _dotT(h, m2_ref[...]) + bm2_ref[...]
    o_ref[...] = 1.0 / (1.0 + jnp.exp(-z))


def _rows(width):
    return pl.BlockSpec((BLK, width), lambda i: (i, 0))


def _full(shape):
    return pl.BlockSpec(shape, lambda i: tuple(0 for _ in shape))


def _tc1(p0, p1, d0, d1, x, Wl, Wr, b):
    return pl.pallas_call(
        _tc1_body,
        grid=(NPAD // BLK,),
        in_specs=[
            _rows(D), _rows(D), _rows(D), _rows(D), _rows(D),
            _full((H, D)), _full((H, D)), _full((1, H)),
        ],
        out_specs=_rows(H),
        out_shape=jax.ShapeDtypeStruct((NPAD, H), jnp.float32),
    )(p0, p1, d0, d1, x, Wl, Wr, b.reshape(1, H))


def _tc2(p0, p1, d0, d1, h1, Wl, Wr, b, M1, bm1, M2, bm2):
    return pl.pallas_call(
        _tc2_body,
        grid=(NPAD // BLK,),
        in_specs=[
            _rows(H), _rows(H), _rows(D), _rows(D), _rows(H),
            _full((H, H)), _full((H, H)), _full((1, H)),
            _full((H, H)), _full((1, H)), _full((C, H)), _full((1, C)),
        ],
        out_specs=_rows(C),
        out_shape=jax.ShapeDtypeStruct((NPAD, C), jnp.float32),
    )(p0, p1, d0, d1, h1, Wl, Wr, b.reshape(1, H), M1, bm1.reshape(1, H),
      M2, bm2.reshape(1, C))


def kernel(x, edge_index, W1l, W1r, b1, W2l, W2r, b2, M1, bm1, M2, bm2):
    src = edge_index[0].reshape(NCHUNK, 1, CH)
    dst = edge_index[1].reshape(NCHUNK, 1, CH)
    xp = jnp.concatenate([x, jnp.zeros((NPAD - N, D), jnp.float32)], axis=0)
    p0, p1, d0, d1 = _seg_sum_deg(xp, src, dst)
    h1 = _tc1(p0, p1, d0, d1, xp, W1l, W1r, b1)
    q0, q1 = _seg_sum(h1, src, dst)
    out = _tc2(q0, q1, d0, d1, h1, W2l, W2r, b2, M1, bm1, M2, bm2)
    return out[:N]


# speedup vs baseline: 10.0186x; 1.1999x over previous
import functools

import jax
import jax.numpy as jnp
from jax import lax
from jax.experimental import pallas as pl
from jax.experimental.pallas import tpu as pltpu
from jax.experimental.pallas import tpu_sc as plsc

N = 10000
E = 320000
D = 128
H = 128
C = 64

NC, NS, L = 2, 16, 16
CH = 128
NCHUNK = E // CH
CH_PER_CORE = NCHUNK // NC
TILE_CHUNKS = 78
EXTRA_PER_CORE = CH_PER_CORE - NS * TILE_CHUNKS
NPAD = 10240
RPT = NPAD // NS
DROWS = NPAD // D
SLAB = 13


def _make_seg_sum(with_deg: bool):
    out_type = [jax.ShapeDtypeStruct((NPAD, D), jnp.float32)] * NC
    scratch = [
        pltpu.VMEM((SLAB, 1, CH), jnp.int32),
        pltpu.VMEM((SLAB, 1, CH), jnp.int32),
        pltpu.VMEM((CH, D), jnp.float32),
        pltpu.VMEM((CH, D), jnp.float32),
        pltpu.VMEM_SHARED((NPAD, D), jnp.float32),
        pltpu.SemaphoreType.DMA,
        pltpu.SemaphoreType.DMA,
        pltpu.SemaphoreType.DMA,
        pltpu.SemaphoreType.DMA,
    ]
    if with_deg:
        out_type += [jax.ShapeDtypeStruct((NPAD, D), jnp.float32)] * NC
        scratch += [
            pltpu.VMEM((DROWS, D), jnp.float32),
            pltpu.VMEM((1, DROWS), jnp.int32),
            pltpu.VMEM((DROWS // NS, D), jnp.float32),
            pltpu.VMEM_SHARED((DROWS, D), jnp.float32),
        ]

    mesh = plsc.VectorSubcoreMesh(core_axis_name="c", subcore_axis_name="s")

    @functools.partial(
        pl.kernel, out_type=tuple(out_type), mesh=mesh,
        scratch_types=tuple(scratch),
        compiler_params=pltpu.CompilerParams(needs_layout_passes=False),
    )
    def seg_sum(*refs):
        if with_deg:
            (tbl, src_hbm, dst_hbm, out0, out1, deg0, deg1,
             src_v, dst_v, rows_v, rows_b, acc_sh, sga, sgb, ssa, ssb,
             deg_v, didx_v, degb_v, deg_sh) = refs
        else:
            (tbl, src_hbm, dst_hbm, out0, out1,
             src_v, dst_v, rows_v, rows_b, acc_sh, sga, sgb, ssa, ssb) = refs
        c = lax.axis_index("c")
        s = lax.axis_index("s")
        zvec = jnp.zeros((L,), jnp.float32)

        def zrow(i, _):
            for j in range(D // L):
                rows_v[i, pl.ds(j * L, L)] = zvec
            return 0
        lax.fori_loop(0, CH, zrow, 0)
        r0 = s * RPT
        for k in range(RPT // CH):
            pltpu.sync_copy(rows_v, acc_sh.at[pl.ds(r0 + k * CH, CH)])
        if with_deg:
            def dz(i, _):
                for j in range(D // L):
                    deg_v[i, pl.ds(j * L, L)] = zvec
                return 0
            lax.fori_loop(0, DROWS, dz, 0)
            iota = lax.iota(jnp.int32, L)
            for g in range(DROWS // L):
                didx_v[0, pl.ds(g * L, L)] = iota + g * L
            @pl.when(s == 0)
            def _():
                pltpu.sync_copy(rows_v, deg_sh.at[pl.ds(0, CH)])

        plsc.subcore_barrier()

        base = c * CH_PER_CORE + s * TILE_CHUNKS

        ones16 = jnp.ones((L,), jnp.float32)

        def hist(j):
            if with_deg:
                for g in range(CH // L):
                    dvec = dst_v[j, 0, pl.ds(g * L, L)]
                    row = lax.shift_right_logical(dvec, 7)
                    col = lax.bitwise_and(dvec, 127)
                    plsc.addupdate_scatter(deg_v, [row, col], ones16)

        def start_g(j, buf, gs):
            pltpu.async_copy(tbl.at[src_v.at[j, 0]], buf, gs)

        def wait_g(j, buf, gs):
            pltpu.make_async_copy(tbl.at[src_v.at[j, 0]], buf, gs).wait()

        def start_s(j, buf, ss):
            pltpu.async_copy(buf, acc_sh.at[dst_v.at[j, 0]], ss, add=True)

        def wait_s(j, buf, ss):
            pltpu.make_async_copy(buf, acc_sh.at[dst_v.at[j, 0]], ss).wait()

        for sl in range(TILE_CHUNKS // SLAB):
            pltpu.sync_copy(src_hbm.at[pl.ds(base + sl * SLAB, SLAB)], src_v)
            pltpu.sync_copy(dst_hbm.at[pl.ds(base + sl * SLAB, SLAB)], dst_v)
            start_g(0, rows_v, sga)

            def pair_body(m, _):
                k0 = 2 * m
                wait_g(k0, rows_v, sga)

                @pl.when(k0 + 1 < SLAB)
                def _():
                    start_g(k0 + 1, rows_b, sgb)
                start_s(k0, rows_v, ssa)
                hist(k0)
                wait_s(k0, rows_v, ssa)

                @pl.when(k0 + 1 < SLAB)
                def _():
                    k1 = k0 + 1
                    wait_g(k1, rows_b, sgb)

                    @pl.when(k1 + 1 < SLAB)
                    def _():
                        start_g(k1 + 1, rows_v, sga)
                    start_s(k1, rows_b, ssb)
                    hist(k1)
                    wait_s(k1, rows_b, ssb)
                return 0
            lax.fori_loop(0, (SLAB + 1) // 2, pair_body, 0)

        @pl.when(s < EXTRA_PER_CORE)
        def _():
            eid = c * CH_PER_CORE + NS * TILE_CHUNKS + s
            pltpu.sync_copy(src_hbm.at[eid], src_v.at[0])
            pltpu.sync_copy(dst_hbm.at[eid], dst_v.at[0])
            pltpu.async_copy(tbl.at[src_v.at[0, 0]], rows_v, sga).wait()
            pltpu.sync_copy(rows_v, acc_sh.at[dst_v.at[0, 0]], add=True)
            hist(0)
        if with_deg:
            pltpu.sync_copy(deg_v, deg_sh.at[didx_v.at[0]], add=True)
        plsc.subcore_barrier()

        def writeback(o_ref, d_ref):
            for k in range(RPT // CH):
                rr = r0 + k * CH
                pltpu.sync_copy(acc_sh.at[pl.ds(rr, CH)], rows_v)
                pltpu.sync_copy(rows_v, o_ref.at[pl.ds(rr, CH)])
            if with_deg:
                pltpu.sync_copy(deg_sh.at[pl.ds(s * (DROWS // NS),
                                                DROWS // NS)], degb_v)
                for k in range(RPT // CH):
                    def egrp(m, _):
                        j0 = k * CH + m * L
                        row = lax.shift_right_logical(j0, 7)
                        col = lax.bitwise_and(j0, 127)
                        dvec = degb_v[row, pl.ds(col, L)]
                        for l in range(L):
                            vv = jnp.full((L,), dvec[l], jnp.float32)
                            for g in range(D // L):
                                rows_v[m * L + l, pl.ds(g * L, L)] = vv
                        return 0
                    lax.fori_loop(0, CH // L, egrp, 0)
                    pltpu.sync_copy(
                        rows_v, d_ref.at[pl.ds(r0 + k * CH, CH)])

        @pl.when(c == 0)
        def _():
            writeback(out0, deg0 if with_deg else None)

        @pl.when(c == 1)
        def _():
            writeback(out1, deg1 if with_deg else None)

    return seg_sum


_seg_sum_deg = _make_seg_sum(True)
_seg_sum = _make_seg_sum(False)

BLK = 512


def _dotT(a, w):
    return lax.dot_general(a, w, (((1,), (1,)), ((), ())),
                           preferred_element_type=jnp.float32)


def _mean_block(p0_ref, p1_ref, d0_ref, d1_ref):
    psum = p0_ref[...] + p1_ref[...]
    deg = d0_ref[...] + d1_ref[...]
    return psum / jnp.maximum(deg, 1.0)


def _tc1_body(p0_ref, p1_ref, d0_ref, d1_ref, x_ref, wl_ref, wr_ref, b_ref,
              o_ref):
    mean = _mean_block(p0_ref, p1_ref, d0_ref, d1_ref)
    h = _dotT(mean, wl_ref[...]) + _dotT(x_ref[...], wr_ref[...]) + b_ref[...]
    o_ref[...] = jnp.maximum(h, 0.0)


def _tc2_body(p0_ref, p1_ref, d0_ref, d1_ref, h_ref, wl_ref, wr_ref, b_ref,
              m1_ref, bm1_ref, m2_ref, bm2_ref, o_ref):
    mean = _mean_block(p0_ref, p1_ref, d0_ref, d1_ref)
    h = _dotT(mean, wl_ref[...]) + _dotT(h_ref[...], wr_ref[...]) + b_ref[...]
    h = jnp.maximum(h, 0.0)
    h = jnp.maximum(_dotT(h, m1_ref[...]) + bm1_ref[...], 0.0)
    z = _dotT(h, m2_ref[...]) + bm2_ref[...]
    o_ref[...] = 1.0 / (1.0 + jnp.exp(-z))


def _rows(width):
    return pl.BlockSpec((BLK, width), lambda i: (i, 0))


def _full(shape):
    return pl.BlockSpec(shape, lambda i: tuple(0 for _ in shape))


def _tc1(p0, p1, d0, d1, x, Wl, Wr, b):
    return pl.pallas_call(
        _tc1_body,
        grid=(NPAD // BLK,),
        in_specs=[
            _rows(D), _rows(D), _rows(D), _rows(D), _rows(D),
            _full((H, D)), _full((H, D)), _full((1, H)),
        ],
        out_specs=_rows(H),
        out_shape=jax.ShapeDtypeStruct((NPAD, H), jnp.float32),
    )(p0, p1, d0, d1, x, Wl, Wr, b.reshape(1, H))


def _tc2(p0, p1, d0, d1, h1, Wl, Wr, b, M1, bm1, M2, bm2):
    return pl.pallas_call(
        _tc2_body,
        grid=(NPAD // BLK,),
        in_specs=[
            _rows(H), _rows(H), _rows(D), _rows(D), _rows(H),
            _full((H, H)), _full((H, H)), _full((1, H)),
            _full((H, H)), _full((1, H)), _full((C, H)), _full((1, C)),
        ],
        out_specs=_rows(C),
        out_shape=jax.ShapeDtypeStruct((NPAD, C), jnp.float32),
    )(p0, p1, d0, d1, h1, Wl, Wr, b.reshape(1, H), M1, bm1.reshape(1, H),
      M2, bm2.reshape(1, C))


def kernel(x, edge_index, W1l, W1r, b1, W2l, W2r, b2, M1, bm1, M2, bm2):
    src = edge_index[0].reshape(NCHUNK, 1, CH)
    dst = edge_index[1].reshape(NCHUNK, 1, CH)
    xp = jnp.concatenate([x, jnp.zeros((NPAD - N, D), jnp.float32)], axis=0)
    p0, p1, d0, d1 = _seg_sum_deg(xp, src, dst)
    h1 = _tc1(p0, p1, d0, d1, xp, W1l, W1r, b1)
    q0, q1 = _seg_sum(h1, src, dst)
    out = _tc2(q0, q1, d0, d1, h1, W2l, W2r, b2, M1, bm1, M2, bm2)
    return out[:N]
